# baseline (speedup 1.0000x reference)
import dataclasses
import functools

import jax
import jax.numpy as jnp
from jax.experimental import pallas as pl
from jax.experimental.pallas import tpu as pltpu
from jax.experimental.pallas import tpu_sc as plsc

N = 10000
E = 320000
DIN = 128
DH = 128
NCLASS = 64

NC = 2
NS = 16
L = 16

NT = NC * NS
CHUNK = 50
EPT = E // NT
NCHUNK = EPT // CHUNK
GROUP = 4
NGROUP = NCHUNK // GROUP
WCHUNK = 40
NWCHUNK = N // WCHUNK


def _segsum_sc(h, esd, D):
  mesh = plsc.VectorSubcoreMesh(core_axis_name="c", subcore_axis_name="s")
  cp = pltpu.CompilerParams()
  if "needs_layout_passes" in pltpu.CompilerParams.__dataclass_fields__:
    cp = dataclasses.replace(cp, needs_layout_passes=False)

  @functools.partial(
      pl.kernel,
      out_type=jax.ShapeDtypeStruct((NC, N, D), jnp.float32),
      mesh=mesh,
      scratch_types=[
          pltpu.VMEM((3, GROUP, CHUNK), jnp.int32),
          pltpu.VMEM((3, GROUP, CHUNK), jnp.int32),
          pltpu.VMEM((CHUNK, D), jnp.float32),
          pltpu.VMEM((CHUNK, D), jnp.float32),
          pltpu.VMEM((CHUNK, D), jnp.float32),
          pltpu.VMEM((CHUNK, D), jnp.float32),
          pltpu.VMEM_SHARED((N, D), jnp.float32),
          pltpu.SemaphoreType.DMA,
          pltpu.SemaphoreType.DMA,
          pltpu.SemaphoreType.DMA,
          pltpu.SemaphoreType.DMA,
          pltpu.SemaphoreType.DMA,
          pltpu.SemaphoreType.DMA,
          pltpu.SemaphoreType.DMA,
          pltpu.SemaphoreType.DMA,
      ],
      compiler_params=cp,
  )
  def k(h_hbm, esd_hbm, out_hbm, ibufa_v, ibufb_v, rows0_v, rows1_v, rows2_v,
        rows3_v, acc_sh, gsem0, gsem1, gsem2, gsem3, ssem0, ssem1, ssem2,
        ssem3):
    c = jax.lax.axis_index("c")
    s = jax.lax.axis_index("s")
    wid = c * NS + s

    zv = jnp.zeros((L,), jnp.float32)

    @pl.loop(0, WCHUNK)
    def _(r):
      for j in range(D // L):
        rows0_v[r, pl.ds(j * L, L)] = zv

    @pl.loop(0, NWCHUNK)
    def _(j):
      @pl.when(j % NS == s)
      def _():
        pltpu.sync_copy(rows0_v.at[pl.ds(0, WCHUNK)],
                        acc_sh.at[pl.ds(j * WCHUNK, WCHUNK)])

    plsc.subcore_barrier()

    rows = (rows0_v, rows1_v, rows2_v, rows3_v)
    gsem = (gsem0, gsem1, gsem2, gsem3)
    ssem = (ssem0, ssem1, ssem2, ssem3)

    def group_copy(g, ib):
      pltpu.sync_copy(esd_hbm.at[wid * NGROUP + g], ib)

    def gather_start(slot, ib, j):
      pltpu.async_copy(h_hbm.at[ib.at[0, j]], rows[slot], gsem[slot])

    def gather_wait(slot, ib, j):
      pltpu.make_async_copy(h_hbm.at[ib.at[0, j]], rows[slot],
                            gsem[slot]).wait()

    def scale(slot, ib, j):
      rv = rows[slot]
      wrow = ib.at[2, j]

      @plsc.parallel_loop(0, CHUNK, unroll=5)
      def _(r):
        wvec = plsc.bitcast(
            plsc.load_gather(wrow, [jnp.full((L,), r, jnp.int32)]),
            jnp.float32)
        for jj in range(D // L):
          sl = pl.ds(jj * L, L)
          rv[r, sl] = rv[r, sl] * wvec

    def scatter_start(slot, ib, j):
      pltpu.async_copy(rows[slot], acc_sh.at[ib.at[1, j]], ssem[slot],
                       add=True)

    def scatter_wait(slot, ib, j):
      pltpu.make_async_copy(rows[slot], acc_sh.at[ib.at[1, j]],
                            ssem[slot]).wait()

    group_copy(0, ibufa_v)
    gather_start(0, ibufa_v, 0)
    gather_start(1, ibufa_v, 1)

    @pl.loop(0, NCHUNK // 8)
    def _(u):
      ga = u * 2
      gb = u * 2 + 1
      for j in range(8):
        slot = j % 4
        pslot = (j + 2) % 4
        ib = ibufa_v if j < 4 else ibufb_v
        gather_wait(slot, ib, j % 4)
        scale(slot, ib, j % 4)
        scatter_start(slot, ib, j % 4)

        if j < 2:
          @pl.when(u > 0)
          def _():
            scatter_wait(pslot, ibufb_v, 2 + j)
          if j == 1:
            group_copy(gb, ibufb_v)
          gather_start(pslot, ibufa_v, 2 + j)
        elif j < 6:
          scatter_wait(pslot, ibufa_v if j < 4 else ibufb_v,
                       (j + 2) % 4 if j < 4 else j - 4)
          if j == 5:
            @pl.when(u < NCHUNK // 8 - 1)
            def _():
              group_copy(ga + 2, ibufa_v)
          gather_start(pslot, ibufb_v, j - 2)
        else:
          scatter_wait(pslot, ibufb_v, j - 4)

          @pl.when(u < NCHUNK // 8 - 1)
          def _():
            gather_start(pslot, ibufa_v, j - 6)

    scatter_wait(2, ibufb_v, 2)
    scatter_wait(3, ibufb_v, 3)
    plsc.subcore_barrier()

    @pl.loop(0, NWCHUNK)
    def _(j):
      @pl.when(j % NS == s)
      def _():
        base = j * WCHUNK
        pltpu.sync_copy(acc_sh.at[pl.ds(base, WCHUNK)],
                        out_hbm.at[c, pl.ds(base, WCHUNK)])

  return k(h, esd)


BM = 1000


def _mm_tc(x, W):
  M, K = x.shape
  Kw, Do = W.shape

  def kern(x_ref, w_ref, o_ref):
    o_ref[...] = jnp.dot(x_ref[...], w_ref[...],
                         preferred_element_type=jnp.float32)

  return pl.pallas_call(
      kern,
      grid=(M // BM,),
      in_specs=[
          pl.BlockSpec((BM, K), lambda i: (i, 0)),
          pl.BlockSpec((Kw, Do), lambda i: (0, 0)),
      ],
      out_specs=pl.BlockSpec((BM, Do), lambda i: (i, 0)),
      out_shape=jax.ShapeDtypeStruct((M, Do), jnp.float32),
  )(x, W)


def _fuse_tc(p, b, g, bb, W):
  _, M, D = p.shape
  Dw, Do = W.shape

  def kern(p_ref, b_ref, g_ref, bb_ref, w_ref, o_ref):
    x = p_ref[0] + p_ref[1] + b_ref[...]
    mu = jnp.mean(x, axis=-1, keepdims=True)
    var = jnp.mean(jnp.square(x - mu), axis=-1, keepdims=True)
    x = (x - mu) * jax.lax.rsqrt(var + 1e-5) * g_ref[...] + bb_ref[...]
    x = jnp.maximum(x, 0.0)
    o_ref[...] = jnp.dot(x, w_ref[...], preferred_element_type=jnp.float32)

  return pl.pallas_call(
      kern,
      grid=(M // BM,),
      in_specs=[
          pl.BlockSpec((2, BM, D), lambda i: (0, i, 0)),
          pl.BlockSpec((1, D), lambda i: (0, 0)),
          pl.BlockSpec((1, D), lambda i: (0, 0)),
          pl.BlockSpec((1, D), lambda i: (0, 0)),
          pl.BlockSpec((Dw, Do), lambda i: (0, 0)),
      ],
      out_specs=pl.BlockSpec((BM, Do), lambda i: (i, 0)),
      out_shape=jax.ShapeDtypeStruct((M, Do), jnp.float32),
  )(p, b, g, bb, W)


def _final_tc(p, b):
  _, M, Dp = p.shape
  D = NCLASS

  def kern(p_ref, b_ref, o_ref):
    x = p_ref[0, :, :D] + p_ref[1, :, :D] + b_ref[...]
    m = jnp.max(x, axis=-1, keepdims=True)
    e = jnp.exp(x - m)
    lse = jnp.log(jnp.sum(e, axis=-1, keepdims=True)) + m
    o_ref[...] = x - lse

  return pl.pallas_call(
      kern,
      grid=(M // BM,),
      in_specs=[
          pl.BlockSpec((2, BM, Dp), lambda i: (0, i, 0)),
          pl.BlockSpec((1, D), lambda i: (0, 0)),
      ],
      out_specs=pl.BlockSpec((BM, D), lambda i: (i, 0)),
      out_shape=jax.ShapeDtypeStruct((M, D), jnp.float32),
  )(p, b)


def kernel(feats, edge_index, edge_weight, W1, b1, W2, b2, W3, b3, W4, b4,
           ln_g, ln_b):
  b1r = b1.reshape(1, DH)
  b2r = b2.reshape(1, DH)
  b3r = b3.reshape(1, DH)
  b4r = b4.reshape(1, NCLASS)
  gr = ln_g.reshape(1, DH)
  br = ln_b.reshape(1, DH)

  shp = (NT, NGROUP, GROUP, CHUNK)
  src_idx = edge_index[0].reshape(shp)
  dst_idx = edge_index[1].reshape(shp)
  wbits = jax.lax.bitcast_convert_type(edge_weight, jnp.int32).reshape(shp)
  esd = jnp.stack([src_idx, dst_idx, wbits], axis=2).reshape(
      NT * NGROUP, 3, GROUP, CHUNK)

  h = _mm_tc(feats, W1)
  p = _segsum_sc(h, esd, DH)
  h = _fuse_tc(p, b1r, gr, br, W2)
  p = _segsum_sc(h, esd, DH)
  h = _fuse_tc(p, b2r, gr, br, W3)
  p = _segsum_sc(h, esd, DH)
  W4p = jnp.pad(W4, ((0, 0), (0, DH - NCLASS)))
  h = _fuse_tc(p, b3r, gr, br, W4p)
  p = _segsum_sc(h, esd, DH)
  return _final_tc(p, b4r)


# speedup vs baseline: 1.0403x; 1.0403x over previous
import dataclasses
import functools

import jax
import jax.numpy as jnp
from jax.experimental import pallas as pl
from jax.experimental.pallas import tpu as pltpu
from jax.experimental.pallas import tpu_sc as plsc

N = 10000
E = 320000
DIN = 128
DH = 128
NCLASS = 64

NC = 2
NS = 16
L = 16

NT = NC * NS
CHUNK = 125
EPT = E // NT
NCHUNK = EPT // CHUNK
GROUP = 4
NGROUP = NCHUNK // GROUP
WCHUNK = 80
NWCHUNK = N // WCHUNK


def _segsum_sc(h, esd, D):
  mesh = plsc.VectorSubcoreMesh(core_axis_name="c", subcore_axis_name="s")
  cp = pltpu.CompilerParams()
  if "needs_layout_passes" in pltpu.CompilerParams.__dataclass_fields__:
    cp = dataclasses.replace(cp, needs_layout_passes=False)

  @functools.partial(
      pl.kernel,
      out_type=jax.ShapeDtypeStruct((NC, N, D), jnp.float32),
      mesh=mesh,
      scratch_types=[
          pltpu.VMEM((3, GROUP, CHUNK), jnp.int32),
          pltpu.VMEM((3, GROUP, CHUNK), jnp.int32),
          pltpu.VMEM((CHUNK, D), jnp.float32),
          pltpu.VMEM((CHUNK, D), jnp.float32),
          pltpu.VMEM_SHARED((N, D), jnp.float32),
          pltpu.SemaphoreType.DMA,
          pltpu.SemaphoreType.DMA,
          pltpu.SemaphoreType.DMA,
          pltpu.SemaphoreType.DMA,
      ],
      compiler_params=cp,
  )
  def k(h_hbm, esd_hbm, out_hbm, ibufa_v, ibufb_v, rows0_v, rows1_v, acc_sh,
        gsem0, gsem1, ssem0, ssem1):
    c = jax.lax.axis_index("c")
    s = jax.lax.axis_index("s")
    wid = c * NS + s

    zv = jnp.zeros((L,), jnp.float32)

    @pl.loop(0, WCHUNK)
    def _(r):
      for j in range(D // L):
        rows0_v[r, pl.ds(j * L, L)] = zv

    @pl.loop(0, NWCHUNK)
    def _(j):
      @pl.when(j % NS == s)
      def _():
        pltpu.sync_copy(rows0_v.at[pl.ds(0, WCHUNK)],
                        acc_sh.at[pl.ds(j * WCHUNK, WCHUNK)])

    plsc.subcore_barrier()

    rows = (rows0_v, rows1_v)
    gsem = (gsem0, gsem1)
    ssem = (ssem0, ssem1)

    def group_copy(g, ib):
      pltpu.sync_copy(esd_hbm.at[wid * NGROUP + g], ib)

    def gather_start(slot, ib, j):
      pltpu.async_copy(h_hbm.at[ib.at[0, j]], rows[slot], gsem[slot])

    def gather_wait(slot, ib, j):
      pltpu.make_async_copy(h_hbm.at[ib.at[0, j]], rows[slot],
                            gsem[slot]).wait()

    def scale(slot, ib, j):
      rv = rows[slot]
      wrow = ib.at[2, j]

      @plsc.parallel_loop(0, CHUNK, unroll=5)
      def _(r):
        wvec = plsc.bitcast(
            plsc.load_gather(wrow, [jnp.full((L,), r, jnp.int32)]),
            jnp.float32)
        for jj in range(D // L):
          sl = pl.ds(jj * L, L)
          rv[r, sl] = rv[r, sl] * wvec

    def scatter_start(slot, ib, j):
      pltpu.async_copy(rows[slot], acc_sh.at[ib.at[1, j]], ssem[slot],
                       add=True)

    def scatter_wait(slot, ib, j):
      pltpu.make_async_copy(rows[slot], acc_sh.at[ib.at[1, j]],
                            ssem[slot]).wait()

    def process(ib, j, slot):
      gather_wait(slot, ib, j)
      scale(slot, ib, j)
      scatter_start(slot, ib, j)

    group_copy(0, ibufa_v)
    gather_start(0, ibufa_v, 0)
    gather_start(1, ibufa_v, 1)

    @pl.loop(0, NCHUNK // 8)
    def _(u):
      process(ibufa_v, 0, 0)
      process(ibufa_v, 1, 1)
      scatter_wait(0, ibufa_v, 0)
      gather_start(0, ibufa_v, 2)
      scatter_wait(1, ibufa_v, 1)
      gather_start(1, ibufa_v, 3)
      process(ibufa_v, 2, 0)
      process(ibufa_v, 3, 1)
      scatter_wait(0, ibufa_v, 2)
      scatter_wait(1, ibufa_v, 3)
      group_copy(u * 2 + 1, ibufb_v)
      gather_start(0, ibufb_v, 0)
      gather_start(1, ibufb_v, 1)
      process(ibufb_v, 0, 0)
      process(ibufb_v, 1, 1)
      scatter_wait(0, ibufb_v, 0)
      gather_start(0, ibufb_v, 2)
      scatter_wait(1, ibufb_v, 1)
      gather_start(1, ibufb_v, 3)
      process(ibufb_v, 2, 0)
      process(ibufb_v, 3, 1)

      @pl.when(u < NCHUNK // 8 - 1)
      def _():
        scatter_wait(0, ibufb_v, 2)
        scatter_wait(1, ibufb_v, 3)
        group_copy(u * 2 + 2, ibufa_v)
        gather_start(0, ibufa_v, 0)
        gather_start(1, ibufa_v, 1)

    scatter_wait(0, ibufb_v, 2)
    scatter_wait(1, ibufb_v, 3)
    plsc.subcore_barrier()

    @pl.loop(0, NWCHUNK)
    def _(j):
      @pl.when(j % NS == s)
      def _():
        base = j * WCHUNK
        pltpu.sync_copy(acc_sh.at[pl.ds(base, WCHUNK)],
                        out_hbm.at[c, pl.ds(base, WCHUNK)])

  return k(h, esd)


BM = 1000


def _mm_tc(x, W):
  M, K = x.shape
  Kw, Do = W.shape

  def kern(x_ref, w_ref, o_ref):
    o_ref[...] = jnp.dot(x_ref[...], w_ref[...],
                         preferred_element_type=jnp.float32)

  return pl.pallas_call(
      kern,
      grid=(M // BM,),
      in_specs=[
          pl.BlockSpec((BM, K), lambda i: (i, 0)),
          pl.BlockSpec((Kw, Do), lambda i: (0, 0)),
      ],
      out_specs=pl.BlockSpec((BM, Do), lambda i: (i, 0)),
      out_shape=jax.ShapeDtypeStruct((M, Do), jnp.float32),
  )(x, W)


def _fuse_tc(p, b, g, bb, W):
  _, M, D = p.shape
  Dw, Do = W.shape

  def kern(p_ref, b_ref, g_ref, bb_ref, w_ref, o_ref):
    x = p_ref[0] + p_ref[1] + b_ref[...]
    mu = jnp.mean(x, axis=-1, keepdims=True)
    var = jnp.mean(jnp.square(x - mu), axis=-1, keepdims=True)
    x = (x - mu) * jax.lax.rsqrt(var + 1e-5) * g_ref[...] + bb_ref[...]
    x = jnp.maximum(x, 0.0)
    o_ref[...] = jnp.dot(x, w_ref[...], preferred_element_type=jnp.float32)

  return pl.pallas_call(
      kern,
      grid=(M // BM,),
      in_specs=[
          pl.BlockSpec((2, BM, D), lambda i: (0, i, 0)),
          pl.BlockSpec((1, D), lambda i: (0, 0)),
          pl.BlockSpec((1, D), lambda i: (0, 0)),
          pl.BlockSpec((1, D), lambda i: (0, 0)),
          pl.BlockSpec((Dw, Do), lambda i: (0, 0)),
      ],
      out_specs=pl.BlockSpec((BM, Do), lambda i: (i, 0)),
      out_shape=jax.ShapeDtypeStruct((M, Do), jnp.float32),
  )(p, b, g, bb, W)


def _final_tc(p, b):
  _, M, Dp = p.shape
  D = NCLASS

  def kern(p_ref, b_ref, o_ref):
    x = p_ref[0, :, :D] + p_ref[1, :, :D] + b_ref[...]
    m = jnp.max(x, axis=-1, keepdims=True)
    e = jnp.exp(x - m)
    lse = jnp.log(jnp.sum(e, axis=-1, keepdims=True)) + m
    o_ref[...] = x - lse

  return pl.pallas_call(
      kern,
      grid=(M // BM,),
      in_specs=[
          pl.BlockSpec((2, BM, Dp), lambda i: (0, i, 0)),
          pl.BlockSpec((1, D), lambda i: (0, 0)),
      ],
      out_specs=pl.BlockSpec((BM, D), lambda i: (i, 0)),
      out_shape=jax.ShapeDtypeStruct((M, D), jnp.float32),
  )(p, b)


def kernel(feats, edge_index, edge_weight, W1, b1, W2, b2, W3, b3, W4, b4,
           ln_g, ln_b):
  b1r = b1.reshape(1, DH)
  b2r = b2.reshape(1, DH)
  b3r = b3.reshape(1, DH)
  b4r = b4.reshape(1, NCLASS)
  gr = ln_g.reshape(1, DH)
  br = ln_b.reshape(1, DH)

  shp = (NT, NGROUP, GROUP, CHUNK)
  src_idx = edge_index[0].reshape(shp)
  dst_idx = edge_index[1].reshape(shp)
  wbits = jax.lax.bitcast_convert_type(edge_weight, jnp.int32).reshape(shp)
  esd = jnp.stack([src_idx, dst_idx, wbits], axis=2).reshape(
      NT * NGROUP, 3, GROUP, CHUNK)

  h = _mm_tc(feats, W1)
  p = _segsum_sc(h, esd, DH)
  h = _fuse_tc(p, b1r, gr, br, W2)
  p = _segsum_sc(h, esd, DH)
  h = _fuse_tc(p, b2r, gr, br, W3)
  p = _segsum_sc(h, esd, DH)
  W4p = jnp.pad(W4, ((0, 0), (0, DH - NCLASS)))
  h = _fuse_tc(p, b3r, gr, br, W4p)
  p = _segsum_sc(h, esd, DH)
  return _final_tc(p, b4r)


# speedup vs baseline: 1.2541x; 1.2055x over previous
import dataclasses
import functools

import jax
import jax.numpy as jnp
from jax.experimental import pallas as pl
from jax.experimental.pallas import tpu as pltpu
from jax.experimental.pallas import tpu_sc as plsc

N = 10000
E = 320000
DIN = 128
DH = 128
NCLASS = 64

NC = 2
NS = 16
L = 16

NT = NC * NS
CHUNK = 100
EPT = E // NT
NCHUNK = EPT // CHUNK
WCHUNK = 80
NWCHUNK = N // WCHUNK


def _segsum_sc(h, esd, D):
  mesh = plsc.VectorSubcoreMesh(core_axis_name="c", subcore_axis_name="s")
  cp = pltpu.CompilerParams()
  if "needs_layout_passes" in pltpu.CompilerParams.__dataclass_fields__:
    cp = dataclasses.replace(cp, needs_layout_passes=False)

  @functools.partial(
      pl.kernel,
      out_type=jax.ShapeDtypeStruct((NC, N, D), jnp.float32),
      mesh=mesh,
      scratch_types=[
          pltpu.VMEM((3, CHUNK), jnp.int32),
          pltpu.VMEM((3, CHUNK), jnp.int32),
          pltpu.VMEM((3, CHUNK), jnp.int32),
          pltpu.VMEM((CHUNK, D), jnp.float32),
          pltpu.VMEM((CHUNK, D), jnp.float32),
          pltpu.VMEM((CHUNK, D), jnp.float32),
          pltpu.VMEM_SHARED((N, D), jnp.float32),
          pltpu.SemaphoreType.DMA,
          pltpu.SemaphoreType.DMA,
          pltpu.SemaphoreType.DMA,
          pltpu.SemaphoreType.DMA,
          pltpu.SemaphoreType.DMA,
          pltpu.SemaphoreType.DMA,
      ],
      compiler_params=cp,
  )
  def k(h_hbm, esd_hbm, out_hbm, ibuf0_v, ibuf1_v, ibuf2_v, rows0_v, rows1_v,
        rows2_v, acc_sh, gsem0, gsem1, gsem2, ssem0, ssem1, ssem2):
    c = jax.lax.axis_index("c")
    s = jax.lax.axis_index("s")
    wid = c * NS + s

    zv = jnp.zeros((L,), jnp.float32)

    @pl.loop(0, WCHUNK)
    def _(r):
      for j in range(D // L):
        rows0_v[r, pl.ds(j * L, L)] = zv

    @pl.loop(0, NWCHUNK)
    def _(j):
      @pl.when(j % NS == s)
      def _():
        pltpu.sync_copy(rows0_v.at[pl.ds(0, WCHUNK)],
                        acc_sh.at[pl.ds(j * WCHUNK, WCHUNK)])

    plsc.subcore_barrier()

    rows = (rows0_v, rows1_v, rows2_v)
    ibufs = (ibuf0_v, ibuf1_v, ibuf2_v)
    gsem = (gsem0, gsem1, gsem2)
    ssem = (ssem0, ssem1, ssem2)

    def idx_copy(t, slot):
      pltpu.sync_copy(esd_hbm.at[wid, t], ibufs[slot])

    def gather_start(slot):
      pltpu.async_copy(h_hbm.at[ibufs[slot].at[0]], rows[slot], gsem[slot])

    def gather_wait(slot):
      pltpu.make_async_copy(h_hbm.at[ibufs[slot].at[0]], rows[slot],
                            gsem[slot]).wait()

    def scale(slot):
      rv = rows[slot]
      wrow = ibufs[slot].at[2]

      @plsc.parallel_loop(0, CHUNK, unroll=5)
      def _(r):
        wvec = plsc.bitcast(
            plsc.load_gather(wrow, [jnp.full((L,), r, jnp.int32)]),
            jnp.float32)
        for jj in range(D // L):
          sl = pl.ds(jj * L, L)
          rv[r, sl] = rv[r, sl] * wvec

    def scatter_start(slot):
      pltpu.async_copy(rows[slot], acc_sh.at[ibufs[slot].at[1]], ssem[slot],
                       add=True)

    def scatter_wait(slot):
      pltpu.make_async_copy(rows[slot], acc_sh.at[ibufs[slot].at[1]],
                            ssem[slot]).wait()

    idx_copy(0, 0)
    gather_start(0)
    idx_copy(1, 1)
    gather_start(1)

    @pl.loop(0, NCHUNK // 3)
    def _(u):
      for a in range(3):
        c = u * 3 + a
        nxt = (a + 2) % 3
        gather_wait(a)
        scale(a)
        scatter_start(a)
        if a == 0:
          @pl.when(u > 0)
          def _():
            scatter_wait(nxt)
        else:
          scatter_wait(nxt)

        @pl.when(c < NCHUNK - 2)
        def _():
          idx_copy(c + 2, nxt)
          gather_start(nxt)

    last = (NCHUNK - 1) % 3
    gather_wait(last)
    scale(last)
    scatter_start(last)
    scatter_wait((last + 2) % 3)
    scatter_wait(last)
    plsc.subcore_barrier()

    @pl.loop(0, NWCHUNK)
    def _(j):
      @pl.when(j % NS == s)
      def _():
        base = j * WCHUNK
        pltpu.sync_copy(acc_sh.at[pl.ds(base, WCHUNK)],
                        out_hbm.at[c, pl.ds(base, WCHUNK)])

  return k(h, esd)


BM = 1000


def _mm_tc(x, W):
  M, K = x.shape
  Kw, Do = W.shape

  def kern(x_ref, w_ref, o_ref):
    o_ref[...] = jnp.dot(x_ref[...], w_ref[...],
                         preferred_element_type=jnp.float32)

  return pl.pallas_call(
      kern,
      grid=(M // BM,),
      in_specs=[
          pl.BlockSpec((BM, K), lambda i: (i, 0)),
          pl.BlockSpec((Kw, Do), lambda i: (0, 0)),
      ],
      out_specs=pl.BlockSpec((BM, Do), lambda i: (i, 0)),
      out_shape=jax.ShapeDtypeStruct((M, Do), jnp.float32),
  )(x, W)


def _fuse_tc(p, b, g, bb, W):
  _, M, D = p.shape
  Dw, Do = W.shape

  def kern(p_ref, b_ref, g_ref, bb_ref, w_ref, o_ref):
    x = p_ref[0] + p_ref[1] + b_ref[...]
    mu = jnp.mean(x, axis=-1, keepdims=True)
    var = jnp.mean(jnp.square(x - mu), axis=-1, keepdims=True)
    x = (x - mu) * jax.lax.rsqrt(var + 1e-5) * g_ref[...] + bb_ref[...]
    x = jnp.maximum(x, 0.0)
    o_ref[...] = jnp.dot(x, w_ref[...], preferred_element_type=jnp.float32)

  return pl.pallas_call(
      kern,
      grid=(M // BM,),
      in_specs=[
          pl.BlockSpec((2, BM, D), lambda i: (0, i, 0)),
          pl.BlockSpec((1, D), lambda i: (0, 0)),
          pl.BlockSpec((1, D), lambda i: (0, 0)),
          pl.BlockSpec((1, D), lambda i: (0, 0)),
          pl.BlockSpec((Dw, Do), lambda i: (0, 0)),
      ],
      out_specs=pl.BlockSpec((BM, Do), lambda i: (i, 0)),
      out_shape=jax.ShapeDtypeStruct((M, Do), jnp.float32),
  )(p, b, g, bb, W)


def _final_tc(p, b):
  _, M, Dp = p.shape
  D = NCLASS

  def kern(p_ref, b_ref, o_ref):
    x = p_ref[0, :, :D] + p_ref[1, :, :D] + b_ref[...]
    m = jnp.max(x, axis=-1, keepdims=True)
    e = jnp.exp(x - m)
    lse = jnp.log(jnp.sum(e, axis=-1, keepdims=True)) + m
    o_ref[...] = x - lse

  return pl.pallas_call(
      kern,
      grid=(M // BM,),
      in_specs=[
          pl.BlockSpec((2, BM, Dp), lambda i: (0, i, 0)),
          pl.BlockSpec((1, D), lambda i: (0, 0)),
      ],
      out_specs=pl.BlockSpec((BM, D), lambda i: (i, 0)),
      out_shape=jax.ShapeDtypeStruct((M, D), jnp.float32),
  )(p, b)


def kernel(feats, edge_index, edge_weight, W1, b1, W2, b2, W3, b3, W4, b4,
           ln_g, ln_b):
  b1r = b1.reshape(1, DH)
  b2r = b2.reshape(1, DH)
  b3r = b3.reshape(1, DH)
  b4r = b4.reshape(1, NCLASS)
  gr = ln_g.reshape(1, DH)
  br = ln_b.reshape(1, DH)

  shp = (NT, NCHUNK, CHUNK)
  src_idx = edge_index[0].reshape(shp)
  dst_idx = edge_index[1].reshape(shp)
  wbits = jax.lax.bitcast_convert_type(edge_weight, jnp.int32).reshape(shp)
  esd = jnp.stack([src_idx, dst_idx, wbits], axis=2)

  h = _mm_tc(feats, W1)
  p = _segsum_sc(h, esd, DH)
  h = _fuse_tc(p, b1r, gr, br, W2)
  p = _segsum_sc(h, esd, DH)
  h = _fuse_tc(p, b2r, gr, br, W3)
  p = _segsum_sc(h, esd, DH)
  W4p = jnp.pad(W4, ((0, 0), (0, DH - NCLASS)))
  h = _fuse_tc(p, b3r, gr, br, W4p)
  p = _segsum_sc(h, esd, DH)
  return _final_tc(p, b4r)
